# Initial kernel scaffold; baseline (speedup 1.0000x reference)
import jax
import jax.numpy as jnp
from jax.experimental import pallas as pl


def kernel(x, edge_index, W1, b1, W2, b2, W3, b3, Wc, bc):
    raise NotImplementedError("write your pallas kernel here")


# speedup vs baseline: 54.7388x; 54.7388x over previous
import jax
import jax.numpy as jnp
from jax import lax
from jax.experimental import pallas as pl
from jax.experimental.pallas import tpu as pltpu
from jax.experimental.pallas import tpu_sc as plsc

N = 10000
E = 320000
D = 128

NC = 2
NS = 16
L = 16

NPAD = 10240
ROWS = NPAD // NS
HALF = N // NC
E_SC = E // NC
E_TILE = E_SC // NS
EW = 2000
DEG_TILE = E // NS

OFF_W2 = 1
OFF_W3 = 17
OFF_WC = 25
OFF_B1 = 27
OFF_B2 = 31
OFF_B3 = 35
OFF_BC = 37
NPARAM = 40

_MESH = plsc.VectorSubcoreMesh(core_axis_name="c", subcore_axis_name="s")
_SC_PARAMS = pltpu.CompilerParams(needs_layout_passes=False)

f32 = jnp.float32
i32 = jnp.int32


def _splat(buf, j):
    return plsc.load_gather(buf, [jnp.full((L,), j, i32)])


def _tanh(a):
    t = jnp.exp(-2.0 * jnp.abs(a))
    return jnp.sign(a) * (1.0 - t) / (1.0 + t)


def _rsqrt(d):
    bits = lax.bitcast_convert_type(d, i32)
    y = lax.bitcast_convert_type(jnp.full((L,), 0x5F3759DF, i32) - (bits >> 1),
                                 f32)
    for _ in range(3):
        y = y * (1.5 - 0.5 * d * y * y)
    return y


def _iota():
    return lax.iota(i32, L)


def _fill_ones(onesb):
    def body(i, _):
        onesb[pl.ds(i * L, L)] = jnp.ones((L,), f32)
        return 0
    lax.fori_loop(0, EW // L, body, 0)


def _edge_phase(esrc, edst, dinv_t, g_t, srcb, dstb, msg, accs, c, s, dout):
    ebase = c * E_SC + s * E_TILE

    def win(w, _):
        eb = ebase + w * EW
        pltpu.sync_copy(esrc.at[pl.ds(eb, EW)], srcb)
        pltpu.sync_copy(edst.at[pl.ds(eb, EW)], dstb)

        def step(k, _):
            sl = pl.ds(k * L, L)
            sv = srcb[sl]
            dv = dstb[sl]
            nrm = plsc.load_gather(dinv_t, [sv]) * plsc.load_gather(dinv_t, [dv])
            svf = sv * dout
            for j in range(dout):
                gj = plsc.load_gather(g_t, [svf + j])
                msg[j][sl] = gj * nrm
            return 0

        lax.fori_loop(0, EW // L, step, 0)
        for j in range(dout):
            pltpu.sync_copy(msg[j], accs[j].at[dstb], add=True)
        return 0

    lax.fori_loop(0, E_TILE // EW, win, 0)


def _write_partials(p_out, accs, c, s, dout):
    rbase = s * ROWS
    for j in range(dout):
        pltpu.sync_copy(accs[j].at[pl.ds(rbase, ROWS)],
                        p_out.at[pl.ds((c * dout + j) * NPAD + rbase, ROWS)])


def _k1_body(g1, esrc, edst, dinv_out, p_out,
             deg_s, acc0, acc1, acc2, acc3,
             dinv_t, g_t, srcb, dstb, m0, m1, m2, m3, onesb, stg):
    accs = (acc0, acc1, acc2, acc3)
    msg = (m0, m1, m2, m3)
    c = lax.axis_index("c")
    s = lax.axis_index("s")
    rbase = s * ROWS

    _fill_ones(onesb)
    pltpu.sync_copy(onesb.at[pl.ds(0, ROWS)], deg_s.at[pl.ds(rbase, ROWS)])
    plsc.subcore_barrier()

    def deg_win(w, _):
        base = s * DEG_TILE + w * EW
        pltpu.sync_copy(edst.at[pl.ds(base, EW)], dstb)
        pltpu.sync_copy(onesb, deg_s.at[dstb], add=True)
        return 0

    lax.fori_loop(0, DEG_TILE // EW, deg_win, 0)
    plsc.subcore_barrier()

    pltpu.sync_copy(deg_s.at[pl.ds(rbase, ROWS)], stg)

    def newton(i, _):
        sl = pl.ds(i * L, L)
        stg[sl] = _rsqrt(stg[sl])
        return 0

    lax.fori_loop(0, ROWS // L, newton, 0)
    plsc.subcore_barrier()
    pltpu.sync_copy(stg, deg_s.at[pl.ds(rbase, ROWS)])

    @pl.when(c == 0)
    def _():
        pltpu.sync_copy(stg, dinv_out.at[pl.ds(rbase, ROWS)])

    plsc.subcore_barrier()
    pltpu.sync_copy(deg_s, dinv_t)

    pltpu.sync_copy(g1, g_t.at[pl.ds(0, 4 * N)])

    def self_init(i, _):
        sl = pl.ds(i * L, L)
        pos = rbase + i * L + _iota()
        dv = dinv_t[pl.ds(rbase + i * L, L)]
        w = dv * dv
        own = jnp.logical_and(pos >= c * HALF, pos < (c + 1) * HALF)
        wm = jnp.where(own, w, 0.0)
        posf = pos * 4
        for j in range(4):
            gj = plsc.load_gather(g_t, [posf + j])
            msg[j][sl] = gj * wm
        return 0

    lax.fori_loop(0, ROWS // L, self_init, 0)
    for j in range(4):
        pltpu.sync_copy(msg[j].at[pl.ds(0, ROWS)], accs[j].at[pl.ds(rbase, ROWS)])
    plsc.subcore_barrier()

    _edge_phase(esrc, edst, dinv_t, g_t, srcb, dstb, msg, accs, c, s, 4)
    plsc.subcore_barrier()
    _write_partials(p_out, accs, c, s, 4)


def _make_k1():
    scratch = [
        pltpu.VMEM_SHARED((NPAD,), f32),
        pltpu.VMEM_SHARED((NPAD,), f32),
        pltpu.VMEM_SHARED((NPAD,), f32),
        pltpu.VMEM_SHARED((NPAD,), f32),
        pltpu.VMEM_SHARED((NPAD,), f32),
        pltpu.VMEM((NPAD,), f32),
        pltpu.VMEM((4 * NPAD,), f32),
        pltpu.VMEM((EW,), i32),
        pltpu.VMEM((EW,), i32),
        pltpu.VMEM((EW,), f32),
        pltpu.VMEM((EW,), f32),
        pltpu.VMEM((EW,), f32),
        pltpu.VMEM((EW,), f32),
        pltpu.VMEM((EW,), f32),
        pltpu.VMEM((ROWS,), f32),
    ]
    out_type = (
        jax.ShapeDtypeStruct((NPAD,), f32),
        jax.ShapeDtypeStruct((NC * 4 * NPAD,), f32),
    )
    return pl.kernel(_k1_body, out_type=out_type, mesh=_MESH,
                     scratch_types=scratch, compiler_params=_SC_PARAMS,
                     name="gcn_sc_k1")


def _mid_body(p_in, esrc, edst, dinv_hbm, params, p_out,
              gshared, accs, dinv_t, g_t, par_t, srcb, dstb, msg, pa, pb,
              gstage, *, din, dout, boff, woff):
    c = lax.axis_index("c")
    s = lax.axis_index("s")
    rbase = s * ROWS

    pltpu.sync_copy(dinv_hbm, dinv_t)
    pltpu.sync_copy(params, par_t)
    for j in range(din):
        pltpu.sync_copy(p_in.at[pl.ds(j * NPAD + rbase, ROWS)], pa[j])
        pltpu.sync_copy(p_in.at[pl.ds((din + j) * NPAD + rbase, ROWS)], pb[j])

    def node(i, _):
        sl = pl.ds(i * L, L)
        pos = rbase + i * L + _iota()
        h = []
        for j in range(din):
            a = pa[j][sl] + pb[j][sl] + _splat(par_t, boff + j)
            h.append(_tanh(a))
        dv = dinv_t[pl.ds(rbase + i * L, L)]
        own = jnp.logical_and(pos >= c * HALF, pos < (c + 1) * HALF)
        wm = jnp.where(own, dv * dv, 0.0)
        lpos = (i * L + _iota()) * dout
        for k in range(dout):
            g = h[0] * _splat(par_t, woff + k)
            for j in range(1, din):
                g = g + h[j] * _splat(par_t, woff + j * dout + k)
            plsc.store_scatter(gstage, [lpos + k], g)
            msg[k][sl] = g * wm
        return 0

    lax.fori_loop(0, ROWS // L, node, 0)
    pltpu.sync_copy(gstage, gshared.at[pl.ds(rbase * dout, ROWS * dout)])
    for k in range(dout):
        pltpu.sync_copy(msg[k].at[pl.ds(0, ROWS)], accs[k].at[pl.ds(rbase, ROWS)])
    plsc.subcore_barrier()
    pltpu.sync_copy(gshared, g_t)
    plsc.subcore_barrier()

    _edge_phase(esrc, edst, dinv_t, g_t, srcb, dstb, msg, accs, c, s, dout)
    plsc.subcore_barrier()
    _write_partials(p_out, accs, c, s, dout)


def _mid_wrap(din, dout, boff, woff):
    def wrapped(p_in, esrc, edst, dinv_hbm, params, p_out, *refs):
        gshared = refs[0]
        accs = refs[1:1 + dout]
        k = 1 + dout
        dinv_t, g_t, par_t, srcb, dstb = refs[k:k + 5]
        k += 5
        msg = refs[k:k + dout]
        k += dout
        pa = refs[k:k + din]
        k += din
        pb = refs[k:k + din]
        k += din
        gstage = refs[k]
        _mid_body(p_in, esrc, edst, dinv_hbm, params, p_out,
                  gshared, accs, dinv_t, g_t, par_t, srcb, dstb, msg, pa, pb,
                  gstage, din=din, dout=dout, boff=boff, woff=woff)
    return wrapped


def _make_mid(din, dout, boff, woff, name):
    scratch = [pltpu.VMEM_SHARED((NPAD * dout,), f32)]
    scratch += [pltpu.VMEM_SHARED((NPAD,), f32) for _ in range(dout)]
    scratch += [
        pltpu.VMEM((NPAD,), f32),
        pltpu.VMEM((NPAD * dout,), f32),
        pltpu.VMEM((NPARAM,), f32),
        pltpu.VMEM((EW,), i32),
        pltpu.VMEM((EW,), i32),
    ]
    scratch += [pltpu.VMEM((EW,), f32) for _ in range(dout)]
    scratch += [pltpu.VMEM((ROWS,), f32) for _ in range(din)]
    scratch += [pltpu.VMEM((ROWS,), f32) for _ in range(din)]
    scratch += [pltpu.VMEM((ROWS * dout,), f32)]
    out_type = jax.ShapeDtypeStruct((NC * dout * NPAD,), f32)
    return pl.kernel(_mid_wrap(din, dout, boff, woff),
                     out_type=out_type, mesh=_MESH,
                     scratch_types=scratch, compiler_params=_SC_PARAMS,
                     name=name)


K4_ROWS = 320
K4_SHORT = HALF - (NS - 1) * K4_ROWS


def _k4_body(p_in, params, out_hbm, h3_hbm,
             par_t, pa0, pa1, pb0, pb1, hstage, ostage):
    c = lax.axis_index("c")
    s = lax.axis_index("s")
    base = c * HALF + s * K4_ROWS

    pltpu.sync_copy(params, par_t)
    pltpu.sync_copy(p_in.at[pl.ds(0 * NPAD + base, K4_ROWS)], pa0)
    pltpu.sync_copy(p_in.at[pl.ds(1 * NPAD + base, K4_ROWS)], pa1)
    pltpu.sync_copy(p_in.at[pl.ds(2 * NPAD + base, K4_ROWS)], pb0)
    pltpu.sync_copy(p_in.at[pl.ds(3 * NPAD + base, K4_ROWS)], pb1)

    def node(i, _):
        sl = pl.ds(i * L, L)
        a0 = pa0[sl] + pb0[sl] + _splat(par_t, OFF_B3 + 0)
        a1 = pa1[sl] + pb1[sl] + _splat(par_t, OFF_B3 + 1)
        h0 = _tanh(a0)
        h1 = _tanh(a1)
        pos2 = (i * L + _iota()) * 2
        plsc.store_scatter(hstage, [pos2], h0)
        plsc.store_scatter(hstage, [pos2 + 1], h1)
        o = (h0 * _splat(par_t, OFF_WC) + h1 * _splat(par_t, OFF_WC + 1)
             + _splat(par_t, OFF_BC))
        ostage[sl] = o
        return 0

    lax.fori_loop(0, K4_ROWS // L, node, 0)

    @pl.when(s < NS - 1)
    def _():
        pltpu.sync_copy(hstage, h3_hbm.at[pl.ds(2 * base, 2 * K4_ROWS)])
        pltpu.sync_copy(ostage, out_hbm.at[pl.ds(base, K4_ROWS)])

    @pl.when(s == NS - 1)
    def _():
        pltpu.sync_copy(hstage.at[pl.ds(0, 2 * K4_SHORT)],
                        h3_hbm.at[pl.ds(2 * base, 2 * K4_SHORT)])
        pltpu.sync_copy(ostage.at[pl.ds(0, K4_SHORT)],
                        out_hbm.at[pl.ds(base, K4_SHORT)])


def _make_k4():
    scratch = [
        pltpu.VMEM((NPARAM,), f32),
        pltpu.VMEM((K4_ROWS,), f32),
        pltpu.VMEM((K4_ROWS,), f32),
        pltpu.VMEM((K4_ROWS,), f32),
        pltpu.VMEM((K4_ROWS,), f32),
        pltpu.VMEM((2 * K4_ROWS,), f32),
        pltpu.VMEM((K4_ROWS,), f32),
    ]
    out_type = (
        jax.ShapeDtypeStruct((N,), f32),
        jax.ShapeDtypeStruct((2 * N,), f32),
    )
    return pl.kernel(_k4_body, out_type=out_type, mesh=_MESH,
                     scratch_types=scratch, compiler_params=_SC_PARAMS,
                     name="gcn_sc_k4")


def _mm_body(x_ref, w_ref, o_ref):
    o_ref[...] = jnp.dot(x_ref[...], w_ref[...], preferred_element_type=f32)


def _tc_matmul(x, w1):
    return pl.pallas_call(
        _mm_body,
        grid=(10,),
        in_specs=[
            pl.BlockSpec((N // 10, D), lambda i: (i, 0)),
            pl.BlockSpec((D, 4), lambda i: (0, 0)),
        ],
        out_specs=pl.BlockSpec((N // 10, 4), lambda i: (i, 0)),
        out_shape=jax.ShapeDtypeStruct((N, 4), f32),
    )(x, w1)


def kernel(x, edge_index, W1, b1, W2, b2, W3, b3, Wc, bc):
    ei = edge_index.astype(i32)
    esrc, edst = ei[0], ei[1]
    params = jnp.concatenate([
        jnp.zeros((1,), f32),
        W2.reshape(-1), W3.reshape(-1), Wc.reshape(-1),
        b1, b2, b3, bc, jnp.zeros((NPARAM - 38,), f32),
    ])
    g1 = _tc_matmul(x, W1).reshape(-1)
    dinv, p1 = _make_k1()(g1, esrc, edst)
    p2 = _make_mid(4, 4, OFF_B1, OFF_W2, "gcn_sc_k2")(p1, esrc, edst, dinv,
                                                      params)
    p3 = _make_mid(4, 2, OFF_B2, OFF_W3, "gcn_sc_k3")(p2, esrc, edst, dinv,
                                                      params)
    out, h3 = _make_k4()(p3, params)
    return (out.reshape(N, 1), h3.reshape(N, 2))


# speedup vs baseline: 63.6954x; 1.1636x over previous
import jax
import jax.numpy as jnp
from jax import lax
from jax.experimental import pallas as pl
from jax.experimental.pallas import tpu as pltpu
from jax.experimental.pallas import tpu_sc as plsc

N = 10000
E = 320000
D = 128

NC = 2
NS = 16
L = 16

NPAD = 10240
ROWS = NPAD // NS
HALF = N // NC
E_SC = E // NC
E_TILE = E_SC // NS
EW = 2000
NW = E_TILE // EW
NB = 3
DEG_TILE = E // NS
NDW = DEG_TILE // EW

OFF_W2 = 1
OFF_W3 = 17
OFF_WC = 25
OFF_B1 = 27
OFF_B2 = 31
OFF_B3 = 35
OFF_BC = 37
NPARAM = 40

_MESH = plsc.VectorSubcoreMesh(core_axis_name="c", subcore_axis_name="s")
_SC_PARAMS = pltpu.CompilerParams(needs_layout_passes=False)

f32 = jnp.float32
i32 = jnp.int32


def _splat(buf, j):
    return plsc.load_gather(buf, [jnp.full((L,), j, i32)])


def _tanh(a):
    t = jnp.exp(-2.0 * jnp.abs(a))
    return jnp.sign(a) * (1.0 - t) / (1.0 + t)


def _rsqrt(d):
    bits = lax.bitcast_convert_type(d, i32)
    y = lax.bitcast_convert_type(jnp.full((L,), 0x5F3759DF, i32) - (bits >> 1),
                                 f32)
    for _ in range(3):
        y = y * (1.5 - 0.5 * d * y * y)
    return y


def _iota():
    return lax.iota(i32, L)


def _k0_body(edst, dinv_out, deg_s, dstb0, dstb1, onesb, stg, sem):
    c = lax.axis_index("c")
    s = lax.axis_index("s")
    rbase = s * ROWS
    dstb = (dstb0, dstb1)

    def fill(i, _):
        onesb[pl.ds(i * L, L)] = jnp.ones((L,), f32)
        return 0
    lax.fori_loop(0, EW // L, fill, 0)
    pltpu.sync_copy(onesb.at[pl.ds(0, ROWS)], deg_s.at[pl.ds(rbase, ROWS)])
    plsc.subcore_barrier()

    tbase = s * DEG_TILE
    ind = [None] * NDW
    ind[0] = pltpu.async_copy(edst.at[pl.ds(tbase, EW)], dstb[0], sem)
    for w in range(NDW):
        if w + 1 < NDW:
            ind[w + 1] = pltpu.async_copy(
                edst.at[pl.ds(tbase + (w + 1) * EW, EW)], dstb[(w + 1) % 2],
                sem)
        ind[w].wait()
        pltpu.sync_copy(onesb, deg_s.at[dstb[w % 2]], add=True)
    plsc.subcore_barrier()

    pltpu.sync_copy(deg_s.at[pl.ds(rbase, ROWS)], stg)

    def newton(i, _):
        sl = pl.ds(i * L, L)
        stg[sl] = _rsqrt(stg[sl])
        return 0

    lax.fori_loop(0, ROWS // L, newton, 0)

    @pl.when(c == 0)
    def _():
        pltpu.sync_copy(stg, dinv_out.at[pl.ds(rbase, ROWS)])


def _make_k0():
    scratch = [
        pltpu.VMEM_SHARED((NPAD,), f32),
        pltpu.VMEM((EW,), i32),
        pltpu.VMEM((EW,), i32),
        pltpu.VMEM((EW,), f32),
        pltpu.VMEM((ROWS,), f32),
        pltpu.SemaphoreType.DMA,
    ]
    return pl.kernel(_k0_body, out_type=jax.ShapeDtypeStruct((NPAD,), f32),
                     mesh=_MESH, scratch_types=scratch,
                     compiler_params=_SC_PARAMS, name="gcn_sc_k0")


def _edge_pipeline(esrc, edst, accs, srcb, dstb, normb, msg, sem_in, sem_out,
                   c, s, dout, g_t, dinv_t=None, norm_out=None, norm_in=None):
    ebase = c * E_SC + s * E_TILE
    first_layer = norm_out is not None
    ind = [None] * NW
    outd = [None] * NW

    def start_in(w):
        b = w % NB
        eb = ebase + w * EW
        ds = [pltpu.async_copy(esrc.at[pl.ds(eb, EW)], srcb[b], sem_in),
              pltpu.async_copy(edst.at[pl.ds(eb, EW)], dstb[b], sem_in)]
        if norm_in is not None:
            ds.append(pltpu.async_copy(norm_in.at[pl.ds(eb, EW)], normb[b],
                                       sem_in))
        return ds

    ind[0] = start_in(0)

    for w in range(NW):
        b = w % NB
        if w + 1 < NW:
            ind[w + 1] = start_in(w + 1)
        for d0 in ind[w]:
            d0.wait()

        def step(k, _):
            sl = pl.ds(k * L, L)
            sv = srcb[b][sl]
            if first_layer:
                dv = dstb[b][sl]
                nrm = (plsc.load_gather(dinv_t, [sv])
                       * plsc.load_gather(dinv_t, [dv]))
                normb[b][sl] = nrm
            else:
                nrm = normb[b][sl]
            svf = sv * dout
            for j in range(dout):
                gj = plsc.load_gather(g_t, [svf + j])
                msg[b][j][sl] = gj * nrm
            return 0

        lax.fori_loop(0, EW // L, step, 0)
        for j in range(dout):
            pltpu.sync_copy(msg[b][j], accs[j].at[dstb[b]], add=True)
        if first_layer:
            outd[w] = pltpu.async_copy(
                normb[b], norm_out.at[pl.ds(ebase + w * EW, EW)], sem_out)
            if w >= 2:
                outd[w - 2].wait()

    if first_layer:
        for w in (NW - 2, NW - 1):
            outd[w].wait()


def _write_partials(p_out, accs, c, s, dout):
    rbase = s * ROWS
    for j in range(dout):
        pltpu.sync_copy(accs[j].at[pl.ds(rbase, ROWS)],
                        p_out.at[pl.ds((c * dout + j) * NPAD + rbase, ROWS)])


def _k1_body(g1, esrc, edst, dinv, p_out, norm_out,
             gshared, dshared, acc0, acc1, acc2, acc3,
             dinv_t, g_t, srcb0, srcb1, srcb2, dstb0, dstb1, dstb2,
             normb0, normb1, normb2, *rest):
    accs = (acc0, acc1, acc2, acc3)
    srcb = (srcb0, srcb1, srcb2)
    dstb = (dstb0, dstb1, dstb2)
    normb = (normb0, normb1, normb2)
    msg = tuple(tuple(rest[b * 4 + j] for j in range(4)) for b in range(NB))
    sem_in, sem_out = rest[NB * 4:NB * 4 + 2]
    c = lax.axis_index("c")
    s = lax.axis_index("s")
    rbase = s * ROWS

    @pl.when(s == 0)
    def _():
        pltpu.sync_copy(g1, gshared)
        pltpu.sync_copy(dinv, dshared)
    plsc.subcore_barrier()
    pltpu.sync_copy(gshared, g_t)
    pltpu.sync_copy(dshared, dinv_t)

    def self_init(i, _):
        sl = pl.ds(i * L, L)
        pos = rbase + i * L + _iota()
        dv = dinv_t[pl.ds(rbase + i * L, L)]
        w = dv * dv
        own = jnp.logical_and(pos >= c * HALF, pos < (c + 1) * HALF)
        wm = jnp.where(own, w, 0.0)
        posf = jnp.minimum(pos, N - 1) * 4
        for j in range(4):
            gj = plsc.load_gather(g_t, [posf + j])
            msg[0][j][sl] = gj * wm
        return 0

    lax.fori_loop(0, ROWS // L, self_init, 0)
    for j in range(4):
        pltpu.sync_copy(msg[0][j].at[pl.ds(0, ROWS)],
                        accs[j].at[pl.ds(rbase, ROWS)])
    plsc.subcore_barrier()

    _edge_pipeline(esrc, edst, accs, srcb, dstb, normb, msg, sem_in, sem_out,
                   c, s, 4, g_t, dinv_t=dinv_t, norm_out=norm_out)
    plsc.subcore_barrier()
    _write_partials(p_out, accs, c, s, 4)


def _make_k1():
    scratch = [
        pltpu.VMEM_SHARED((4 * N,), f32),
        pltpu.VMEM_SHARED((NPAD,), f32),
        pltpu.VMEM_SHARED((NPAD,), f32),
        pltpu.VMEM_SHARED((NPAD,), f32),
        pltpu.VMEM_SHARED((NPAD,), f32),
        pltpu.VMEM_SHARED((NPAD,), f32),
        pltpu.VMEM((NPAD,), f32),
        pltpu.VMEM((4 * N,), f32),
    ]
    scratch += [pltpu.VMEM((EW,), i32) for _ in range(3)]
    scratch += [pltpu.VMEM((EW,), i32) for _ in range(3)]
    scratch += [pltpu.VMEM((EW,), f32) for _ in range(3)]
    scratch += [pltpu.VMEM((EW,), f32) for _ in range(NB * 4)]
    scratch += [
        pltpu.SemaphoreType.DMA,
        pltpu.SemaphoreType.DMA,
    ]
    out_type = (
        jax.ShapeDtypeStruct((NC * 4 * NPAD,), f32),
        jax.ShapeDtypeStruct((E,), f32),
    )
    return pl.kernel(_k1_body, out_type=out_type, mesh=_MESH,
                     scratch_types=scratch, compiler_params=_SC_PARAMS,
                     name="gcn_sc_k1")


def _mid_body(p_in, esrc, edst, norm, dinv, params, p_out, refs,
              *, din, dout, boff, woff):
    gshared = refs[0]
    accs = refs[1:1 + dout]
    k = 1 + dout
    g_t, par_t, dinv_sl = refs[k:k + 3]
    k += 3
    srcb = refs[k:k + 3]
    dstb = refs[k + 3:k + 6]
    normb = refs[k + 6:k + 9]
    k += 9
    msg = tuple(tuple(refs[k + b * dout + j] for j in range(dout))
                for b in range(NB))
    k += NB * dout
    gstage = refs[k]
    sem_in, sem_out = refs[k + 1:k + 3]
    pa = [gstage.at[pl.ds((dout + j) * ROWS, ROWS)] for j in range(din)]
    pb = [gstage.at[pl.ds((dout + din + j) * ROWS, ROWS)] for j in range(din)]
    gst = gstage.at[pl.ds(0, ROWS * dout)]

    c = lax.axis_index("c")
    s = lax.axis_index("s")
    rbase = s * ROWS

    pltpu.sync_copy(params, par_t)
    pltpu.sync_copy(dinv.at[pl.ds(rbase, ROWS)], dinv_sl)
    for j in range(din):
        pltpu.sync_copy(p_in.at[pl.ds(j * NPAD + rbase, ROWS)], pa[j])
        pltpu.sync_copy(p_in.at[pl.ds((din + j) * NPAD + rbase, ROWS)], pb[j])

    def node(i, _):
        sl = pl.ds(i * L, L)
        pos = rbase + i * L + _iota()
        h = []
        for j in range(din):
            a = pa[j][sl] + pb[j][sl] + _splat(par_t, boff + j)
            h.append(_tanh(a))
        dv = dinv_sl[sl]
        own = jnp.logical_and(pos >= c * HALF, pos < (c + 1) * HALF)
        wm = jnp.where(own, dv * dv, 0.0)
        lpos = (i * L + _iota()) * dout
        for kk in range(dout):
            g = h[0] * _splat(par_t, woff + kk)
            for j in range(1, din):
                g = g + h[j] * _splat(par_t, woff + j * dout + kk)
            plsc.store_scatter(gst, [lpos + kk], g)
            msg[0][kk][sl] = g * wm
        return 0

    lax.fori_loop(0, ROWS // L, node, 0)
    pltpu.sync_copy(gst, gshared.at[pl.ds(rbase * dout, ROWS * dout)])
    for kk in range(dout):
        pltpu.sync_copy(msg[0][kk].at[pl.ds(0, ROWS)],
                        accs[kk].at[pl.ds(rbase, ROWS)])
    plsc.subcore_barrier()
    pltpu.sync_copy(gshared, g_t)
    plsc.subcore_barrier()

    _edge_pipeline(esrc, edst, accs, srcb, dstb, normb, msg, sem_in, sem_out,
                   c, s, dout, g_t, norm_in=norm)
    plsc.subcore_barrier()
    _write_partials(p_out, accs, c, s, dout)


def _make_mid(din, dout, boff, woff, name):
    def wrapped(p_in, esrc, edst, norm, dinv, params, p_out, *refs):
        _mid_body(p_in, esrc, edst, norm, dinv, params, p_out, refs,
                  din=din, dout=dout, boff=boff, woff=woff)

    scratch = [pltpu.VMEM_SHARED((NPAD * dout,), f32)]
    scratch += [pltpu.VMEM_SHARED((NPAD,), f32) for _ in range(dout)]
    scratch += [
        pltpu.VMEM((NPAD * dout,), f32),
        pltpu.VMEM((NPARAM,), f32),
        pltpu.VMEM((ROWS,), f32),
    ]
    scratch += [pltpu.VMEM((EW,), i32) for _ in range(3)]
    scratch += [pltpu.VMEM((EW,), i32) for _ in range(3)]
    scratch += [pltpu.VMEM((EW,), f32) for _ in range(3)]
    scratch += [pltpu.VMEM((EW,), f32) for _ in range(NB * dout)]
    scratch += [
        pltpu.VMEM(((dout + 2 * din) * ROWS,), f32),
        pltpu.SemaphoreType.DMA,
        pltpu.SemaphoreType.DMA,
    ]
    out_type = jax.ShapeDtypeStruct((NC * dout * NPAD,), f32)
    return pl.kernel(wrapped, out_type=out_type, mesh=_MESH,
                     scratch_types=scratch, compiler_params=_SC_PARAMS,
                     name=name)


K4_ROWS = 320
K4_SHORT = HALF - (NS - 1) * K4_ROWS


def _k4_body(p_in, params, out_hbm, h3_hbm,
             par_t, pa0, pa1, pb0, pb1, hstage, ostage):
    c = lax.axis_index("c")
    s = lax.axis_index("s")
    base = c * HALF + s * K4_ROWS

    pltpu.sync_copy(params, par_t)
    pltpu.sync_copy(p_in.at[pl.ds(0 * NPAD + base, K4_ROWS)], pa0)
    pltpu.sync_copy(p_in.at[pl.ds(1 * NPAD + base, K4_ROWS)], pa1)
    pltpu.sync_copy(p_in.at[pl.ds(2 * NPAD + base, K4_ROWS)], pb0)
    pltpu.sync_copy(p_in.at[pl.ds(3 * NPAD + base, K4_ROWS)], pb1)

    def node(i, _):
        sl = pl.ds(i * L, L)
        a0 = pa0[sl] + pb0[sl] + _splat(par_t, OFF_B3 + 0)
        a1 = pa1[sl] + pb1[sl] + _splat(par_t, OFF_B3 + 1)
        h0 = _tanh(a0)
        h1 = _tanh(a1)
        pos2 = (i * L + _iota()) * 2
        plsc.store_scatter(hstage, [pos2], h0)
        plsc.store_scatter(hstage, [pos2 + 1], h1)
        o = (h0 * _splat(par_t, OFF_WC) + h1 * _splat(par_t, OFF_WC + 1)
             + _splat(par_t, OFF_BC))
        ostage[sl] = o
        return 0

    lax.fori_loop(0, K4_ROWS // L, node, 0)

    @pl.when(s < NS - 1)
    def _():
        pltpu.sync_copy(hstage, h3_hbm.at[pl.ds(2 * base, 2 * K4_ROWS)])
        pltpu.sync_copy(ostage, out_hbm.at[pl.ds(base, K4_ROWS)])

    @pl.when(s == NS - 1)
    def _():
        pltpu.sync_copy(hstage.at[pl.ds(0, 2 * K4_SHORT)],
                        h3_hbm.at[pl.ds(2 * base, 2 * K4_SHORT)])
        pltpu.sync_copy(ostage.at[pl.ds(0, K4_SHORT)],
                        out_hbm.at[pl.ds(base, K4_SHORT)])


def _make_k4():
    scratch = [
        pltpu.VMEM((NPARAM,), f32),
        pltpu.VMEM((K4_ROWS,), f32),
        pltpu.VMEM((K4_ROWS,), f32),
        pltpu.VMEM((K4_ROWS,), f32),
        pltpu.VMEM((K4_ROWS,), f32),
        pltpu.VMEM((2 * K4_ROWS,), f32),
        pltpu.VMEM((K4_ROWS,), f32),
    ]
    out_type = (
        jax.ShapeDtypeStruct((N,), f32),
        jax.ShapeDtypeStruct((2 * N,), f32),
    )
    return pl.kernel(_k4_body, out_type=out_type, mesh=_MESH,
                     scratch_types=scratch, compiler_params=_SC_PARAMS,
                     name="gcn_sc_k4")


def _mm_body(x_ref, w_ref, o_ref):
    o_ref[...] = jnp.dot(x_ref[...], w_ref[...], preferred_element_type=f32)


def _tc_matmul(x, w1):
    return pl.pallas_call(
        _mm_body,
        grid=(10,),
        in_specs=[
            pl.BlockSpec((N // 10, D), lambda i: (i, 0)),
            pl.BlockSpec((D, 4), lambda i: (0, 0)),
        ],
        out_specs=pl.BlockSpec((N // 10, 4), lambda i: (i, 0)),
        out_shape=jax.ShapeDtypeStruct((N, 4), f32),
    )(x, w1)


def kernel(x, edge_index, W1, b1, W2, b2, W3, b3, Wc, bc):
    ei = edge_index.astype(i32)
    esrc, edst = ei[0], ei[1]
    params = jnp.concatenate([
        jnp.zeros((1,), f32),
        W2.reshape(-1), W3.reshape(-1), Wc.reshape(-1),
        b1, b2, b3, bc, jnp.zeros((NPARAM - 38,), f32),
    ])
    dinv = _make_k0()(edst)
    g1 = _tc_matmul(x, W1).reshape(-1)
    p1, norm = _make_k1()(g1, esrc, edst, dinv)
    p2 = _make_mid(4, 4, OFF_B1, OFF_W2, "gcn_sc_k2")(p1, esrc, edst, norm,
                                                      dinv, params)
    p3 = _make_mid(4, 2, OFF_B2, OFF_W3, "gcn_sc_k3")(p2, esrc, edst, norm,
                                                      dinv, params)
    out, h3 = _make_k4()(p3, params)
    return (out.reshape(N, 1), h3.reshape(N, 2))


# speedup vs baseline: 74.5078x; 1.1698x over previous
import jax
import jax.numpy as jnp
from jax import lax
from jax.experimental import pallas as pl
from jax.experimental.pallas import tpu as pltpu
from jax.experimental.pallas import tpu_sc as plsc

N = 10000
E = 320000
D = 128

NC = 2
NS = 16
L = 16

NPAD = 10240
ROWS = NPAD // NS
HALF = N // NC
E_SC = E // NC
E_TILE = E_SC // NS
EW = 2000
NW = E_TILE // EW
NB = 3
DEG_TILE = E // NS
NDW = DEG_TILE // EW

OFF_W2 = 1
OFF_W3 = 17
OFF_WC = 25
OFF_B1 = 27
OFF_B2 = 31
OFF_B3 = 35
OFF_BC = 37
NPARAM = 40

_MESH = plsc.VectorSubcoreMesh(core_axis_name="c", subcore_axis_name="s")
_SC_PARAMS = pltpu.CompilerParams(needs_layout_passes=False)

f32 = jnp.float32
i32 = jnp.int32


def _splat(buf, j):
    return plsc.load_gather(buf, [jnp.full((L,), j, i32)])


def _tanh(a):
    t = jnp.exp(-2.0 * jnp.abs(a))
    return jnp.sign(a) * (1.0 - t) / (1.0 + t)


def _rsqrt(d):
    bits = lax.bitcast_convert_type(d, i32)
    y = lax.bitcast_convert_type(jnp.full((L,), 0x5F3759DF, i32) - (bits >> 1),
                                 f32)
    for _ in range(3):
        y = y * (1.5 - 0.5 * d * y * y)
    return y


def _iota():
    return lax.iota(i32, L)


def _edge_pipeline(esrc, edst, accs, srcb, dstb, normb, msg, sem_in, sem_out,
                   c, s, dout, g_t, dinv_t=None, norm_out=None, norm_in=None):
    ebase = c * E_SC + s * E_TILE
    first_layer = norm_out is not None
    ind = [None] * NW
    outd = [None] * NW

    def start_in(w):
        b = w % NB
        eb = ebase + w * EW
        ds = [pltpu.async_copy(esrc.at[pl.ds(eb, EW)], srcb[b], sem_in),
              pltpu.async_copy(edst.at[pl.ds(eb, EW)], dstb[b], sem_in)]
        if norm_in is not None:
            ds.append(pltpu.async_copy(norm_in.at[pl.ds(eb, EW)], normb[b],
                                       sem_in))
        return ds

    ind[0] = start_in(0)

    for w in range(NW):
        b = w % NB
        if w + 1 < NW:
            ind[w + 1] = start_in(w + 1)
        for d0 in ind[w]:
            d0.wait()

        @plsc.parallel_loop(0, EW // L, unroll=4)
        def _(k):
            sl = pl.ds(k * L, L)
            sv = srcb[b][sl]
            if first_layer:
                dv = dstb[b][sl]
                nrm = (plsc.load_gather(dinv_t, [sv])
                       * plsc.load_gather(dinv_t, [dv]))
                normb[b][sl] = nrm
            else:
                nrm = normb[b][sl]
            svf = sv * dout
            for j in range(dout):
                gj = plsc.load_gather(g_t, [svf + j])
                msg[b][j][sl] = gj * nrm

        for j in range(dout):
            pltpu.sync_copy(msg[b][j], accs[j].at[dstb[b]], add=True)
        if first_layer:
            outd[w] = pltpu.async_copy(
                normb[b], norm_out.at[pl.ds(ebase + w * EW, EW)], sem_out)
            if w >= 2:
                outd[w - 2].wait()

    if first_layer:
        for w in (NW - 2, NW - 1):
            outd[w].wait()


def _write_partials(p_out, accs, c, s, dout):
    rbase = s * ROWS
    for j in range(dout):
        pltpu.sync_copy(accs[j].at[pl.ds(rbase, ROWS)],
                        p_out.at[pl.ds((c * dout + j) * NPAD + rbase, ROWS)])


def _k1_body(g1, esrc, edst, p_out, norm_out, dinv_out,
             gshared, acc0, acc1, acc2, acc3,
             dinv_t, g_t, srcb0, srcb1, srcb2, dstb0, dstb1, dstb2,
             normb0, normb1, normb2, *rest):
    accs = (acc0, acc1, acc2, acc3)
    srcb = (srcb0, srcb1, srcb2)
    dstb = (dstb0, dstb1, dstb2)
    normb = (normb0, normb1, normb2)
    msg = tuple(tuple(rest[b * 4 + j] for j in range(4)) for b in range(NB))
    onesb, stg = rest[NB * 4:NB * 4 + 2]
    sem_in, sem_out = rest[NB * 4 + 2:NB * 4 + 4]
    c = lax.axis_index("c")
    s = lax.axis_index("s")
    rbase = s * ROWS

    gdesc = pltpu.make_async_copy(g1, gshared, sem_out)

    @pl.when(s == 0)
    def _():
        gdesc.start()

    @plsc.parallel_loop(0, EW // L, unroll=4)
    def _(i):
        onesb[pl.ds(i * L, L)] = jnp.ones((L,), f32)

    pltpu.sync_copy(onesb.at[pl.ds(0, ROWS)], acc0.at[pl.ds(rbase, ROWS)])
    plsc.subcore_barrier()

    tbase = s * DEG_TILE
    ind = [None] * NDW
    ind[0] = pltpu.async_copy(edst.at[pl.ds(tbase, EW)], dstb[0], sem_in)
    for w in range(NDW):
        if w + 1 < NDW:
            ind[w + 1] = pltpu.async_copy(
                edst.at[pl.ds(tbase + (w + 1) * EW, EW)], dstb[(w + 1) % 2],
                sem_in)
        ind[w].wait()
        pltpu.sync_copy(onesb, acc0.at[dstb[w % 2]], add=True)
    plsc.subcore_barrier()

    pltpu.sync_copy(acc0.at[pl.ds(rbase, ROWS)], stg)

    @plsc.parallel_loop(0, ROWS // L, unroll=4)
    def _(i):
        sl = pl.ds(i * L, L)
        stg[sl] = _rsqrt(stg[sl])

    @pl.when(c == 0)
    def _():
        pltpu.sync_copy(stg, dinv_out.at[pl.ds(rbase, ROWS)])

    pltpu.sync_copy(stg, acc0.at[pl.ds(rbase, ROWS)])

    @pl.when(s == 0)
    def _():
        gdesc.wait()

    plsc.subcore_barrier()
    pltpu.sync_copy(acc0, dinv_t)
    pltpu.sync_copy(gshared, g_t)
    plsc.subcore_barrier()

    @plsc.parallel_loop(0, ROWS // L, unroll=4)
    def _(i):
        sl = pl.ds(i * L, L)
        pos = rbase + i * L + _iota()
        dv = dinv_t[pl.ds(rbase + i * L, L)]
        w = dv * dv
        own = jnp.logical_and(pos >= c * HALF, pos < (c + 1) * HALF)
        wm = jnp.where(own, w, 0.0)
        posf = jnp.minimum(pos, N - 1) * 4
        for j in range(4):
            gj = plsc.load_gather(g_t, [posf + j])
            msg[0][j][sl] = gj * wm

    for j in range(4):
        pltpu.sync_copy(msg[0][j].at[pl.ds(0, ROWS)],
                        accs[j].at[pl.ds(rbase, ROWS)])
    plsc.subcore_barrier()

    _edge_pipeline(esrc, edst, accs, srcb, dstb, normb, msg, sem_in, sem_out,
                   c, s, 4, g_t, dinv_t=dinv_t, norm_out=norm_out)
    plsc.subcore_barrier()
    _write_partials(p_out, accs, c, s, 4)


def _make_k1():
    scratch = [
        pltpu.VMEM_SHARED((4 * N,), f32),
        pltpu.VMEM_SHARED((NPAD,), f32),
        pltpu.VMEM_SHARED((NPAD,), f32),
        pltpu.VMEM_SHARED((NPAD,), f32),
        pltpu.VMEM_SHARED((NPAD,), f32),
        pltpu.VMEM((NPAD,), f32),
        pltpu.VMEM((4 * N,), f32),
    ]
    scratch += [pltpu.VMEM((EW,), i32) for _ in range(3)]
    scratch += [pltpu.VMEM((EW,), i32) for _ in range(3)]
    scratch += [pltpu.VMEM((EW,), f32) for _ in range(3)]
    scratch += [pltpu.VMEM((EW,), f32) for _ in range(NB * 4)]
    scratch += [
        pltpu.VMEM((EW,), f32),
        pltpu.VMEM((ROWS,), f32),
        pltpu.SemaphoreType.DMA,
        pltpu.SemaphoreType.DMA,
    ]
    out_type = (
        jax.ShapeDtypeStruct((NC * 4 * NPAD,), f32),
        jax.ShapeDtypeStruct((E,), f32),
        jax.ShapeDtypeStruct((NPAD,), f32),
    )
    return pl.kernel(_k1_body, out_type=out_type, mesh=_MESH,
                     scratch_types=scratch, compiler_params=_SC_PARAMS,
                     name="gcn_sc_k1")


def _mid_body(p_in, esrc, edst, norm, dinv, params, p_out, refs,
              *, din, dout, boff, woff):
    gshared = refs[0]
    accs = refs[1:1 + dout]
    k = 1 + dout
    g_t, par_t, dinv_sl = refs[k:k + 3]
    k += 3
    srcb = refs[k:k + 3]
    dstb = refs[k + 3:k + 6]
    normb = refs[k + 6:k + 9]
    k += 9
    msg = tuple(tuple(refs[k + b * dout + j] for j in range(dout))
                for b in range(NB))
    k += NB * dout
    gstage = refs[k]
    sem_in, sem_out = refs[k + 1:k + 3]
    pa = [gstage.at[pl.ds((dout + j) * ROWS, ROWS)] for j in range(din)]
    pb = [gstage.at[pl.ds((dout + din + j) * ROWS, ROWS)] for j in range(din)]
    gst = gstage.at[pl.ds(0, ROWS * dout)]

    c = lax.axis_index("c")
    s = lax.axis_index("s")
    rbase = s * ROWS

    pltpu.sync_copy(params, par_t)
    pltpu.sync_copy(dinv.at[pl.ds(rbase, ROWS)], dinv_sl)
    for j in range(din):
        pltpu.sync_copy(p_in.at[pl.ds(j * NPAD + rbase, ROWS)], pa[j])
        pltpu.sync_copy(p_in.at[pl.ds((din + j) * NPAD + rbase, ROWS)], pb[j])

    @plsc.parallel_loop(0, ROWS // L, unroll=2)
    def _(i):
        sl = pl.ds(i * L, L)
        pos = rbase + i * L + _iota()
        h = []
        for j in range(din):
            a = pa[j][sl] + pb[j][sl] + _splat(par_t, boff + j)
            h.append(_tanh(a))
        dv = dinv_sl[sl]
        own = jnp.logical_and(pos >= c * HALF, pos < (c + 1) * HALF)
        wm = jnp.where(own, dv * dv, 0.0)
        lpos = (i * L + _iota()) * dout
        for kk in range(dout):
            g = h[0] * _splat(par_t, woff + kk)
            for j in range(1, din):
                g = g + h[j] * _splat(par_t, woff + j * dout + kk)
            plsc.store_scatter(gst, [lpos + kk], g)
            msg[0][kk][sl] = g * wm

    pltpu.sync_copy(gst, gshared.at[pl.ds(rbase * dout, ROWS * dout)])
    for kk in range(dout):
        pltpu.sync_copy(msg[0][kk].at[pl.ds(0, ROWS)],
                        accs[kk].at[pl.ds(rbase, ROWS)])
    plsc.subcore_barrier()
    pltpu.sync_copy(gshared, g_t)
    plsc.subcore_barrier()

    _edge_pipeline(esrc, edst, accs, srcb, dstb, normb, msg, sem_in, sem_out,
                   c, s, dout, g_t, norm_in=norm)
    plsc.subcore_barrier()
    _write_partials(p_out, accs, c, s, dout)


def _make_mid(din, dout, boff, woff, name):
    def wrapped(p_in, esrc, edst, norm, dinv, params, p_out, *refs):
        _mid_body(p_in, esrc, edst, norm, dinv, params, p_out, refs,
                  din=din, dout=dout, boff=boff, woff=woff)

    scratch = [pltpu.VMEM_SHARED((NPAD * dout,), f32)]
    scratch += [pltpu.VMEM_SHARED((NPAD,), f32) for _ in range(dout)]
    scratch += [
        pltpu.VMEM((NPAD * dout,), f32),
        pltpu.VMEM((NPARAM,), f32),
        pltpu.VMEM((ROWS,), f32),
    ]
    scratch += [pltpu.VMEM((EW,), i32) for _ in range(3)]
    scratch += [pltpu.VMEM((EW,), i32) for _ in range(3)]
    scratch += [pltpu.VMEM((EW,), f32) for _ in range(3)]
    scratch += [pltpu.VMEM((EW,), f32) for _ in range(NB * dout)]
    scratch += [
        pltpu.VMEM(((dout + 2 * din) * ROWS,), f32),
        pltpu.SemaphoreType.DMA,
        pltpu.SemaphoreType.DMA,
    ]
    out_type = jax.ShapeDtypeStruct((NC * dout * NPAD,), f32)
    return pl.kernel(wrapped, out_type=out_type, mesh=_MESH,
                     scratch_types=scratch, compiler_params=_SC_PARAMS,
                     name=name)


K4_ROWS = 320
K4_SHORT = HALF - (NS - 1) * K4_ROWS


def _k4_body(p_in, params, out_hbm, h3_hbm,
             par_t, pa0, pa1, pb0, pb1, hstage, ostage):
    c = lax.axis_index("c")
    s = lax.axis_index("s")
    base = c * HALF + s * K4_ROWS

    pltpu.sync_copy(params, par_t)
    pltpu.sync_copy(p_in.at[pl.ds(0 * NPAD + base, K4_ROWS)], pa0)
    pltpu.sync_copy(p_in.at[pl.ds(1 * NPAD + base, K4_ROWS)], pa1)
    pltpu.sync_copy(p_in.at[pl.ds(2 * NPAD + base, K4_ROWS)], pb0)
    pltpu.sync_copy(p_in.at[pl.ds(3 * NPAD + base, K4_ROWS)], pb1)

    @plsc.parallel_loop(0, K4_ROWS // L, unroll=2)
    def _(i):
        sl = pl.ds(i * L, L)
        a0 = pa0[sl] + pb0[sl] + _splat(par_t, OFF_B3 + 0)
        a1 = pa1[sl] + pb1[sl] + _splat(par_t, OFF_B3 + 1)
        h0 = _tanh(a0)
        h1 = _tanh(a1)
        pos2 = (i * L + _iota()) * 2
        plsc.store_scatter(hstage, [pos2], h0)
        plsc.store_scatter(hstage, [pos2 + 1], h1)
        o = (h0 * _splat(par_t, OFF_WC) + h1 * _splat(par_t, OFF_WC + 1)
             + _splat(par_t, OFF_BC))
        ostage[sl] = o

    @pl.when(s < NS - 1)
    def _():
        pltpu.sync_copy(hstage, h3_hbm.at[pl.ds(2 * base, 2 * K4_ROWS)])
        pltpu.sync_copy(ostage, out_hbm.at[pl.ds(base, K4_ROWS)])

    @pl.when(s == NS - 1)
    def _():
        pltpu.sync_copy(hstage.at[pl.ds(0, 2 * K4_SHORT)],
                        h3_hbm.at[pl.ds(2 * base, 2 * K4_SHORT)])
        pltpu.sync_copy(ostage.at[pl.ds(0, K4_SHORT)],
                        out_hbm.at[pl.ds(base, K4_SHORT)])


def _make_k4():
    scratch = [
        pltpu.VMEM((NPARAM,), f32),
        pltpu.VMEM((K4_ROWS,), f32),
        pltpu.VMEM((K4_ROWS,), f32),
        pltpu.VMEM((K4_ROWS,), f32),
        pltpu.VMEM((K4_ROWS,), f32),
        pltpu.VMEM((2 * K4_ROWS,), f32),
        pltpu.VMEM((K4_ROWS,), f32),
    ]
    out_type = (
        jax.ShapeDtypeStruct((N,), f32),
        jax.ShapeDtypeStruct((2 * N,), f32),
    )
    return pl.kernel(_k4_body, out_type=out_type, mesh=_MESH,
                     scratch_types=scratch, compiler_params=_SC_PARAMS,
                     name="gcn_sc_k4")


def _mm_body(x_ref, w_ref, o_ref):
    o_ref[...] = jnp.dot(x_ref[...], w_ref[...], preferred_element_type=f32)


def _tc_matmul(x, w1):
    return pl.pallas_call(
        _mm_body,
        grid=(10,),
        in_specs=[
            pl.BlockSpec((N // 10, D), lambda i: (i, 0)),
            pl.BlockSpec((D, 4), lambda i: (0, 0)),
        ],
        out_specs=pl.BlockSpec((N // 10, 4), lambda i: (i, 0)),
        out_shape=jax.ShapeDtypeStruct((N, 4), f32),
    )(x, w1)


def kernel(x, edge_index, W1, b1, W2, b2, W3, b3, Wc, bc):
    ei = edge_index.astype(i32)
    esrc, edst = ei[0], ei[1]
    params = jnp.concatenate([
        jnp.zeros((1,), f32),
        W2.reshape(-1), W3.reshape(-1), Wc.reshape(-1),
        b1, b2, b3, bc, jnp.zeros((NPARAM - 38,), f32),
    ])
    g1 = _tc_matmul(x, W1).reshape(-1)
    p1, norm, dinv = _make_k1()(g1, esrc, edst)
    p2 = _make_mid(4, 4, OFF_B1, OFF_W2, "gcn_sc_k2")(p1, esrc, edst, norm,
                                                      dinv, params)
    p3 = _make_mid(4, 2, OFF_B2, OFF_W3, "gcn_sc_k3")(p2, esrc, edst, norm,
                                                      dinv, params)
    out, h3 = _make_k4()(p3, params)
    return (out.reshape(N, 1), h3.reshape(N, 2))
